# baseline (speedup 1.0000x reference)
import functools

import jax
import jax.numpy as jnp
from jax import lax
from jax.experimental import pallas as pl
from jax.experimental.pallas import tpu as pltpu
from jax.experimental.pallas import tpu_sc as plsc

_VOCAB = 1000000
_DIM = 64
_BATCH = 16384

_INFO = plsc.get_sparse_core_info()
_NC = _INFO.num_cores
_NS = _INFO.num_subcores
_NW = _NC * _NS
_L = _INFO.num_lanes
_N_PER_W = _BATCH // _NW
_GROUPS = _N_PER_W // _L
_ICHUNK = 128
_NCHUNK = _N_PER_W // _ICHUNK
_WPR = _DIM // 2
_PROWS = _N_PER_W // 2


def _rsqrt_newton(x):
    i = lax.bitcast_convert_type(x, jnp.int32)
    i = jnp.int32(0x5F3759DF) - lax.shift_right_logical(i, 1)
    y = lax.bitcast_convert_type(i, jnp.float32)
    xh = x * jnp.float32(0.5)
    for _ in range(3):
        y = y * (jnp.float32(1.5) - xh * y * y)
    return y


def _unpack2(w):
    lo = lax.bitcast_convert_type(lax.shift_left(w, 16), jnp.float32)
    hi = lax.bitcast_convert_type(
        lax.bitwise_and(w, jnp.int32(-65536)), jnp.float32
    )
    return lo, hi


def _process_table(table_pk, idx_ref, pidx, rows, f32out, sem):
    def pidx_body(g, carry):
        v = lax.shift_right_logical(idx_ref[pl.ds(g * _L, _L)], 2)
        pidx[g // 8, pl.ds((g % 8) * _L, _L)] = v
        return carry

    lax.fori_loop(0, _GROUPS, pidx_body, 0)

    for k in range(_NCHUNK):
        pltpu.async_copy(
            table_pk.at[pidx.at[k]],
            rows.at[pl.ds(k * _ICHUNK, _ICHUNK)],
            sem,
        )
    pltpu.make_async_copy(table_pk.at[pl.ds(0, _N_PER_W)], rows, sem).wait()

    iota = lax.broadcasted_iota(jnp.int32, (_L,), 0)

    def group_body(g, carry):
        slot = g * _L + iota
        v = idx_ref[pl.ds(g * _L, _L)]
        wb = lax.shift_left(v & 3, 5)

        def ss_body(m, acc):
            w = plsc.load_gather(rows, [slot, wb + m])
            lo, hi = _unpack2(w)
            return acc + lo * lo + hi * hi

        ss = lax.fori_loop(0, _WPR, ss_body, jnp.zeros((_L,), jnp.float32),
                           unroll=4)
        inv = _rsqrt_newton(ss)
        orow = lax.shift_right_logical(slot, 1)
        ocol = lax.shift_left(slot & 1, 6)

        def scale_body(m, carry2):
            w = plsc.load_gather(rows, [slot, wb + m])
            lo, hi = _unpack2(w)
            c = ocol + 2 * m
            plsc.store_scatter(f32out, [orow, c], lo * inv)
            plsc.store_scatter(f32out, [orow, c + 1], hi * inv)
            return carry2

        lax.fori_loop(0, _WPR, scale_body, 0, unroll=4)
        return carry

    lax.fori_loop(0, _GROUPS, group_body, 0)


@functools.partial(
    pl.kernel,
    out_type=jax.ShapeDtypeStruct((2, _BATCH // 2, 128), jnp.float32),
    mesh=plsc.VectorSubcoreMesh(core_axis_name="c", subcore_axis_name="s"),
    compiler_params=pltpu.CompilerParams(needs_layout_passes=False),
    scratch_types=[
        pltpu.VMEM((_N_PER_W,), jnp.int32),
        pltpu.VMEM((_N_PER_W,), jnp.int32),
        pltpu.VMEM((_NCHUNK, _ICHUNK), jnp.int32),
        pltpu.VMEM((_N_PER_W, 128), jnp.int32),
        pltpu.VMEM((_PROWS, 128), jnp.float32),
        pltpu.SemaphoreType.DMA,
        pltpu.SemaphoreType.DMA,
    ],
)
def _sc_kernel(in_data, out_data, in_pk, out_pk, out,
               idx0, idx1, pidx, rows, f32out, sem, osem):
    wid = lax.axis_index("s") * _NC + lax.axis_index("c")
    base = wid * _N_PER_W

    pltpu.sync_copy(in_data.at[pl.ds(base, _N_PER_W)], idx0)
    pltpu.sync_copy(out_data.at[pl.ds(base, _N_PER_W)], idx1)

    _process_table(in_pk, idx0, pidx, rows, f32out, sem)
    pltpu.async_copy(
        f32out, out.at[0, pl.ds(wid * _PROWS, _PROWS)], osem
    ).wait()

    _process_table(out_pk, idx1, pidx, rows, f32out, sem)
    pltpu.async_copy(
        f32out, out.at[1, pl.ds(wid * _PROWS, _PROWS)], osem
    ).wait()


def _pack(table):
    xb = lax.bitcast_convert_type(table, jnp.int32)
    lo = lax.shift_right_logical(xb[:, 0::2], 16)
    hi = lax.bitwise_and(xb[:, 1::2], jnp.int32(-65536))
    return (lo | hi).reshape(_VOCAB // 4, 128)


def kernel(in_data, out_data, in_table, out_table):
    res = _sc_kernel(
        in_data.astype(jnp.int32), out_data.astype(jnp.int32),
        _pack(in_table), _pack(out_table),
    )
    return res.reshape(2, _BATCH, _DIM)


# speedup vs baseline: 3.2011x; 3.2011x over previous
import functools

import jax
import jax.numpy as jnp
from jax import lax
from jax.experimental import pallas as pl
from jax.experimental.pallas import tpu as pltpu
from jax.experimental.pallas import tpu_sc as plsc

_VOCAB = 1000000
_DIM = 64
_BATCH = 16384

_INFO = plsc.get_sparse_core_info()
_NC = _INFO.num_cores
_NS = _INFO.num_subcores
_NW = _NC * _NS
_L = _INFO.num_lanes
_N_PER_W = _BATCH // _NW
_GROUPS = _N_PER_W // _L
_ICHUNK = 128
_NCHUNK = _N_PER_W // _ICHUNK
_WPR = _DIM
_PROWS = _N_PER_W // 2


def _rsqrt_newton(x):
    i = lax.bitcast_convert_type(x, jnp.int32)
    i = jnp.int32(0x5F3759DF) - lax.shift_right_logical(i, 1)
    y = lax.bitcast_convert_type(i, jnp.float32)
    xh = x * jnp.float32(0.5)
    for _ in range(3):
        y = y * (jnp.float32(1.5) - xh * y * y)
    return y


def _unpack2(w):
    lo = lax.bitcast_convert_type(lax.shift_left(w, 16), jnp.float32)
    hi = lax.bitcast_convert_type(
        lax.bitwise_and(w, jnp.int32(-65536)), jnp.float32
    )
    return lo, hi


def _process_table(table_pk, idx_ref, pidx, rows, f32out, sem):
    def pidx_body(g, carry):
        v = lax.shift_right_logical(idx_ref[pl.ds(g * _L, _L)], 1)
        pidx[g // 8, pl.ds((g % 8) * _L, _L)] = v
        return carry

    lax.fori_loop(0, _GROUPS, pidx_body, 0)

    for k in range(_NCHUNK):
        pltpu.async_copy(
            table_pk.at[pidx.at[k]],
            rows.at[pl.ds(k * _ICHUNK, _ICHUNK)],
            sem,
        )
    pltpu.make_async_copy(table_pk.at[pl.ds(0, _N_PER_W)], rows, sem).wait()

    iota = lax.broadcasted_iota(jnp.int32, (_L,), 0)

    def group_body(g, carry):
        slot = g * _L + iota
        v = idx_ref[pl.ds(g * _L, _L)]
        wb = lax.shift_left(v & 1, 6)

        def ss_body(m, acc):
            x = plsc.load_gather(rows, [slot, wb + m])
            return acc + x * x

        ss = lax.fori_loop(0, _WPR, ss_body, jnp.zeros((_L,), jnp.float32),
                           unroll=4)
        inv = _rsqrt_newton(ss)
        orow = lax.shift_right_logical(slot, 1)
        ocol = lax.shift_left(slot & 1, 6)

        def scale_body(m, carry2):
            x = plsc.load_gather(rows, [slot, wb + m])
            plsc.store_scatter(f32out, [orow, ocol + m], x * inv)
            return carry2

        lax.fori_loop(0, _WPR, scale_body, 0, unroll=4)
        return carry

    lax.fori_loop(0, _GROUPS, group_body, 0)


@functools.partial(
    pl.kernel,
    out_type=jax.ShapeDtypeStruct((2, _BATCH // 2, 128), jnp.float32),
    mesh=plsc.VectorSubcoreMesh(core_axis_name="c", subcore_axis_name="s"),
    compiler_params=pltpu.CompilerParams(needs_layout_passes=False),
    scratch_types=[
        pltpu.VMEM((_N_PER_W,), jnp.int32),
        pltpu.VMEM((_N_PER_W,), jnp.int32),
        pltpu.VMEM((_NCHUNK, _ICHUNK), jnp.int32),
        pltpu.VMEM((_N_PER_W, 128), jnp.float32),
        pltpu.VMEM((_PROWS, 128), jnp.float32),
        pltpu.SemaphoreType.DMA,
        pltpu.SemaphoreType.DMA,
    ],
)
def _sc_kernel(in_data, out_data, in_pk, out_pk, out,
               idx0, idx1, pidx, rows, f32out, sem, osem):
    wid = lax.axis_index("s") * _NC + lax.axis_index("c")
    base = wid * _N_PER_W

    pltpu.sync_copy(in_data.at[pl.ds(base, _N_PER_W)], idx0)
    pltpu.sync_copy(out_data.at[pl.ds(base, _N_PER_W)], idx1)

    _process_table(in_pk, idx0, pidx, rows, f32out, sem)
    pltpu.async_copy(
        f32out, out.at[0, pl.ds(wid * _PROWS, _PROWS)], osem
    ).wait()

    _process_table(out_pk, idx1, pidx, rows, f32out, sem)
    pltpu.async_copy(
        f32out, out.at[1, pl.ds(wid * _PROWS, _PROWS)], osem
    ).wait()


def _pack(table):
    return table.reshape(_VOCAB // 2, 128)


def kernel(in_data, out_data, in_table, out_table):
    res = _sc_kernel(
        in_data.astype(jnp.int32), out_data.astype(jnp.int32),
        _pack(in_table), _pack(out_table),
    )
    return res.reshape(2, _BATCH, _DIM)


# speedup vs baseline: 4.7983x; 1.4990x over previous
import functools

import jax
import jax.numpy as jnp
from jax import lax
from jax.experimental import pallas as pl
from jax.experimental.pallas import tpu as pltpu
from jax.experimental.pallas import tpu_sc as plsc

_VOCAB = 1000000
_DIM = 64
_BATCH = 16384

_INFO = plsc.get_sparse_core_info()
_NC = _INFO.num_cores
_NS = _INFO.num_subcores
_NW = _NC * _NS
_L = _INFO.num_lanes
_N_PER_W = _BATCH // _NW
_GROUPS = _N_PER_W // _L
_PACK = 128 // _DIM
_PROWS = _N_PER_W // _PACK


def _rsqrt_newton(x):
    i = lax.bitcast_convert_type(x, jnp.int32)
    i = jnp.int32(0x5F3759DF) - lax.shift_right_logical(i, 1)
    y = lax.bitcast_convert_type(i, jnp.float32)
    xh = x * jnp.float32(0.5)
    for _ in range(3):
        y = y * (jnp.float32(1.5) - xh * y * y)
    return y


def _normalize_rows(rows_ref):
    iota = lax.broadcasted_iota(jnp.int32, (_L,), 0)

    def group_body(g, carry):
        rr = g * _L + iota
        prow = lax.shift_right_logical(rr, 1)
        pcol = lax.shift_left(rr & 1, 6)

        def ss_body(c, acc):
            x = plsc.load_gather(rows_ref, [prow, pcol + c])
            return acc + x * x

        ss = lax.fori_loop(0, _DIM, ss_body, jnp.zeros((_L,), jnp.float32),
                           unroll=8)
        inv = _rsqrt_newton(ss)

        def scale_body(c, carry2):
            x = plsc.load_gather(rows_ref, [prow, pcol + c])
            plsc.store_scatter(rows_ref, [prow, pcol + c], x * inv)
            return carry2

        lax.fori_loop(0, _DIM, scale_body, 0, unroll=8)
        return carry

    lax.fori_loop(0, _GROUPS, group_body, 0)


def _issue_row_gathers(table, idx_ref, rows_ref, sem):
    def g_body(g, carry):
        iv = idx_ref[pl.ds(g * _L, _L)]
        for j in range(_L):
            r = g * _L + j
            pltpu.async_copy(
                table.at[iv[j]],
                rows_ref.at[r >> 1, pl.ds((r & 1) * _DIM, _DIM)],
                sem,
            )
        return carry

    lax.fori_loop(0, _GROUPS, g_body, 0)


@functools.partial(
    pl.kernel,
    out_type=jax.ShapeDtypeStruct((2, _BATCH // _PACK, 128), jnp.float32),
    mesh=plsc.VectorSubcoreMesh(core_axis_name="c", subcore_axis_name="s"),
    compiler_params=pltpu.CompilerParams(needs_layout_passes=False),
    scratch_types=[
        pltpu.VMEM((_N_PER_W,), jnp.int32),
        pltpu.VMEM((_N_PER_W,), jnp.int32),
        pltpu.VMEM((_PROWS, 128), jnp.float32),
        pltpu.VMEM((_PROWS, 128), jnp.float32),
        pltpu.SemaphoreType.DMA,
        pltpu.SemaphoreType.DMA,
        pltpu.SemaphoreType.DMA,
    ],
)
def _sc_kernel(in_data, out_data, in_table, out_table, out,
               idx0, idx1, rows0, rows1, sem0, sem1, osem):
    wid = lax.axis_index("s") * _NC + lax.axis_index("c")
    base = wid * _N_PER_W
    pbase = wid * _PROWS

    pltpu.sync_copy(in_data.at[pl.ds(base, _N_PER_W)], idx0)
    pltpu.sync_copy(out_data.at[pl.ds(base, _N_PER_W)], idx1)

    _issue_row_gathers(in_table, idx0, rows0, sem0)
    _issue_row_gathers(out_table, idx1, rows1, sem1)

    pltpu.make_async_copy(out.at[0, pl.ds(0, _PROWS)], rows0, sem0).wait()
    _normalize_rows(rows0)
    o0 = pltpu.async_copy(rows0, out.at[0, pl.ds(pbase, _PROWS)], osem)

    pltpu.make_async_copy(out.at[1, pl.ds(0, _PROWS)], rows1, sem1).wait()
    _normalize_rows(rows1)
    o1 = pltpu.async_copy(rows1, out.at[1, pl.ds(pbase, _PROWS)], osem)

    o0.wait()
    o1.wait()


def kernel(in_data, out_data, in_table, out_table):
    packed = _sc_kernel(
        in_data.astype(jnp.int32), out_data.astype(jnp.int32),
        in_table, out_table,
    )
    return packed.reshape(2, _BATCH, _DIM)
